# Initial kernel scaffold; baseline (speedup 1.0000x reference)
import jax
import jax.numpy as jnp
from jax.experimental import pallas as pl


def kernel(landmarks, We, be, W1, b1, W2, b2, W3, b3, Wf, bf, Wc, bc):
    raise NotImplementedError("write your pallas kernel here")


# speedup vs baseline: 38.3979x; 38.3979x over previous
import functools

import jax
import jax.numpy as jnp
import numpy as np
from jax.experimental import pallas as pl

_N = 468
_NPAD = 512


def _a_hat_np(n: int, npad: int) -> np.ndarray:
    src, dst = [], []
    for i in range(n):
        for j in range(i + 1, min(i + 10, n)):
            src += [i, j]
            dst += [j, i]
    src = np.concatenate([np.array(src, np.int64), np.arange(n)])
    dst = np.concatenate([np.array(dst, np.int64), np.arange(n)])
    deg = np.zeros((n,), np.float64)
    np.add.at(deg, dst, 1.0)
    dinv = np.where(deg > 0, deg ** -0.5, 0.0)
    a = np.zeros((npad, npad), np.float64)
    np.add.at(a, (dst, src), dinv[src] * dinv[dst])
    return a.astype(np.float32)


def _gnn_body(coords_ref, we_ref, be_ref, w1_ref, b1_ref, w2_ref, b2_ref,
              w3_ref, b3_ref, a_ref, mask_ref, pooled_ref):
    a_hat = a_ref[...]
    c = coords_ref[0]
    x = jnp.dot(c, we_ref[...], preferred_element_type=jnp.float32)
    x = x + be_ref[...]
    h = jnp.dot(x, w1_ref[...], preferred_element_type=jnp.float32)
    x = jax.nn.relu(jnp.dot(a_hat, h, preferred_element_type=jnp.float32)
                    + b1_ref[...])
    h = jnp.dot(x, w2_ref[...], preferred_element_type=jnp.float32)
    x = jax.nn.relu(jnp.dot(a_hat, h, preferred_element_type=jnp.float32)
                    + b2_ref[...])
    h = jnp.dot(x, w3_ref[...], preferred_element_type=jnp.float32)
    x = jax.nn.relu(jnp.dot(a_hat, h, preferred_element_type=jnp.float32)
                    + b3_ref[...])
    pooled_ref[...] = jnp.sum(x * mask_ref[...], axis=0, keepdims=True)[None]


def _head_body(pooled_ref, wf_ref, bf_ref, wc_ref, bc_ref,
               feats_ref, out_ref):
    f = jax.nn.relu(jnp.dot(pooled_ref[...], wf_ref[...],
                            preferred_element_type=jnp.float32) + bf_ref[...])
    feats_ref[...] = f
    out_ref[...] = jnp.dot(f, wc_ref[...],
                           preferred_element_type=jnp.float32) + bc_ref[...]


@functools.partial(jax.jit, static_argnames=())
def kernel(landmarks, We, be, W1, b1, W2, b2, W3, b3, Wf, bf, Wc, bc):
    bsz = landmarks.shape[0]
    n = landmarks.shape[1] // 3

    coords = landmarks.reshape(bsz, n, 3)
    coords = jnp.pad(coords, ((0, 0), (0, _NPAD - n), (0, 5)))
    we8 = jnp.pad(We, ((0, 5), (0, 0)))

    a_hat = jnp.asarray(_a_hat_np(n, _NPAD))
    mask = jnp.asarray(
        (np.arange(_NPAD) < n).astype(np.float32)[:, None] / n)

    c1 = W1.shape[1]
    c3 = W3.shape[1]
    const = pl.BlockSpec(None, lambda b: (0, 0))

    pooled = pl.pallas_call(
        _gnn_body,
        grid=(bsz,),
        in_specs=[
            pl.BlockSpec((1, _NPAD, 8), lambda b: (b, 0, 0)),
            const, const, const, const, const, const, const, const,
            const, const,
        ],
        out_specs=pl.BlockSpec((1, 1, c1), lambda b: (b, 0, 0)),
        out_shape=jax.ShapeDtypeStruct((bsz, 1, c1), jnp.float32),
    )(coords, we8, be.reshape(1, -1), W1, b1.reshape(1, -1),
      W2, b2.reshape(1, -1), W3, b3.reshape(1, -1), a_hat, mask)

    cf = Wf.shape[1]
    cc = Wc.shape[1]
    feats, out = pl.pallas_call(
        _head_body,
        out_shape=(jax.ShapeDtypeStruct((bsz, cf), jnp.float32),
                   jax.ShapeDtypeStruct((bsz, cc), jnp.float32)),
    )(pooled.reshape(bsz, c3), Wf, bf.reshape(1, -1), Wc, bc.reshape(1, -1))

    return (out.reshape(bsz, 1, cc),
            feats.reshape(bsz, 1, cf),
            pooled)


# speedup vs baseline: 67.8356x; 1.7667x over previous
import functools

import jax
import jax.numpy as jnp
import numpy as np
from jax.experimental import pallas as pl

_N = 468
_NPAD = 512
_G = 4


def _a_hat_np(n: int, npad: int) -> np.ndarray:
    src, dst = [], []
    for i in range(n):
        for j in range(i + 1, min(i + 10, n)):
            src += [i, j]
            dst += [j, i]
    src = np.concatenate([np.array(src, np.int64), np.arange(n)])
    dst = np.concatenate([np.array(dst, np.int64), np.arange(n)])
    deg = np.zeros((n,), np.float64)
    np.add.at(deg, dst, 1.0)
    dinv = np.where(deg > 0, deg ** -0.5, 0.0)
    a = np.zeros((npad, npad), np.float64)
    np.add.at(a, (dst, src), dinv[src] * dinv[dst])
    return a.astype(np.float32)


def _gnn_body(coords_ref, we_ref, be_ref, w1_ref, b1_ref, w2_ref, b2_ref,
              w3_ref, b3_ref, a_ref, mask_ref, pooled_ref):
    a_hat = a_ref[...]
    mask = mask_ref[...]

    def msg(h, b):
        parts = [
            jax.nn.relu(
                jnp.dot(a_hat, h[g * _NPAD:(g + 1) * _NPAD],
                        preferred_element_type=jnp.float32) + b)
            for g in range(_G)
        ]
        return jnp.concatenate(parts, axis=0)

    c = coords_ref[...].reshape(_G * _NPAD, 8)
    x = jnp.dot(c, we_ref[...], preferred_element_type=jnp.float32)
    x = x + be_ref[...]
    h = jnp.dot(x, w1_ref[...], preferred_element_type=jnp.float32)
    x = msg(h, b1_ref[...])
    h = jnp.dot(x, w2_ref[...], preferred_element_type=jnp.float32)
    x = msg(h, b2_ref[...])
    h = jnp.dot(x, w3_ref[...], preferred_element_type=jnp.float32)
    x = msg(h, b3_ref[...])
    for g in range(_G):
        pooled_ref[g] = jnp.sum(x[g * _NPAD:(g + 1) * _NPAD] * mask,
                                axis=0, keepdims=True)


def _head_body(pooled_ref, wf_ref, bf_ref, wc_ref, bc_ref,
               feats_ref, out_ref):
    f = jax.nn.relu(jnp.dot(pooled_ref[...], wf_ref[...],
                            preferred_element_type=jnp.float32) + bf_ref[...])
    feats_ref[...] = f
    out_ref[...] = jnp.dot(f, wc_ref[...],
                           preferred_element_type=jnp.float32) + bc_ref[...]


@functools.partial(jax.jit, static_argnames=())
def kernel(landmarks, We, be, W1, b1, W2, b2, W3, b3, Wf, bf, Wc, bc):
    bsz = landmarks.shape[0]
    n = landmarks.shape[1] // 3

    coords = landmarks.reshape(bsz, n, 3)
    coords = jnp.pad(coords, ((0, 0), (0, _NPAD - n), (0, 5)))
    we8 = jnp.pad(We, ((0, 5), (0, 0)))

    a_hat = jnp.asarray(_a_hat_np(n, _NPAD))
    mask = jnp.asarray(
        (np.arange(_NPAD) < n).astype(np.float32)[:, None] / n)

    c1 = W1.shape[1]
    c3 = W3.shape[1]
    const = pl.BlockSpec(None, lambda b: (0, 0))

    pooled = pl.pallas_call(
        _gnn_body,
        grid=(bsz // _G,),
        in_specs=[
            pl.BlockSpec((_G, _NPAD, 8), lambda b: (b, 0, 0)),
            const, const, const, const, const, const, const, const,
            const, const,
        ],
        out_specs=pl.BlockSpec((_G, 1, c1), lambda b: (b, 0, 0)),
        out_shape=jax.ShapeDtypeStruct((bsz, 1, c1), jnp.float32),
    )(coords, we8, be.reshape(1, -1), W1, b1.reshape(1, -1),
      W2, b2.reshape(1, -1), W3, b3.reshape(1, -1), a_hat, mask)

    cf = Wf.shape[1]
    cc = Wc.shape[1]
    feats, out = pl.pallas_call(
        _head_body,
        out_shape=(jax.ShapeDtypeStruct((bsz, cf), jnp.float32),
                   jax.ShapeDtypeStruct((bsz, cc), jnp.float32)),
    )(pooled.reshape(bsz, c3), Wf, bf.reshape(1, -1), Wc, bc.reshape(1, -1))

    return (out.reshape(bsz, 1, cc),
            feats.reshape(bsz, 1, cf),
            pooled)
